# Initial kernel scaffold; baseline (speedup 1.0000x reference)
import jax
import jax.numpy as jnp
from jax.experimental import pallas as pl


def kernel(hete_1hop, hete_2hop, drug_homo_1hop, drug_homo_2hop, mic_homo_1hop, mic_homo_2hop, x1, x2):
    raise NotImplementedError("write your pallas kernel here")


# speedup vs baseline: 1.4551x; 1.4551x over previous
import functools
import jax
import jax.numpy as jnp
from jax import lax
from jax.experimental import pallas as pl
from jax.experimental.pallas import tpu as pltpu
from jax.experimental.pallas import tpu_sc as plsc

_D = 256
_N_DRUG = 1373
_B = 16384
_NC = 2
_NS = 16
_NW = _NC * _NS
_CB = 32
_BPW = _B // _NW
_NCHUNK = _BPW // _CB
_L = 16


def _body(h1, h2, d1, d2, m1, m2, x1, x2, out,
          idx1_v, idx2_v, idx2h_v,
          b0, b1, b2, b3, b4, b5, b6, b7, sem):
    wid = lax.axis_index("s") * _NC + lax.axis_index("c")

    def chunk(c, _):
        base = wid * _BPW + c * _CB
        pltpu.sync_copy(x1.at[pl.ds(base, _CB)], idx1_v)
        pltpu.sync_copy(x2.at[pl.ds(base, _CB)], idx2_v)
        for j in range(_CB // _L):
            idx2h_v[pl.ds(j * _L, _L)] = idx2_v[pl.ds(j * _L, _L)] + _N_DRUG

        cps = [
            pltpu.async_copy(d1.at[idx1_v], b0, sem),
            pltpu.async_copy(d2.at[idx1_v], b1, sem),
            pltpu.async_copy(h1.at[idx1_v], b2, sem),
            pltpu.async_copy(h2.at[idx1_v], b3, sem),
            pltpu.async_copy(m1.at[idx2_v], b4, sem),
            pltpu.async_copy(m2.at[idx2_v], b5, sem),
            pltpu.async_copy(h1.at[idx2h_v], b6, sem),
            pltpu.async_copy(h2.at[idx2h_v], b7, sem),
        ]
        for cp in cps:
            cp.wait()

        for k, buf in enumerate((b0, b1, b2, b3, b4, b5, b6, b7)):
            pltpu.sync_copy(buf, out.at[pl.ds(base, _CB), pl.ds(k * _D, _D)])
        return ()

    lax.fori_loop(0, _NCHUNK, chunk, (), unroll=False)


@jax.jit
def _run(h1, h2, d1, d2, m1, m2, x1, x2):
    mesh = plsc.VectorSubcoreMesh(core_axis_name="c", subcore_axis_name="s")
    f = pl.kernel(
        _body,
        out_type=jax.ShapeDtypeStruct((_B, 8 * _D), jnp.float32),
        mesh=mesh,
        scratch_types=[
            pltpu.VMEM((_CB,), jnp.int32),
            pltpu.VMEM((_CB,), jnp.int32),
            pltpu.VMEM((_CB,), jnp.int32),
        ] + [pltpu.VMEM((_CB, _D), jnp.float32) for _ in range(8)] + [
            pltpu.SemaphoreType.DMA,
        ],
    )
    return f(h1, h2, d1, d2, m1, m2, x1, x2)


def kernel(hete_1hop, hete_2hop, drug_homo_1hop, drug_homo_2hop,
           mic_homo_1hop, mic_homo_2hop, x1, x2):
    out = _run(hete_1hop, hete_2hop, drug_homo_1hop, drug_homo_2hop,
               mic_homo_1hop, mic_homo_2hop,
               x1.astype(jnp.int32), x2.astype(jnp.int32))
    return out.reshape(_B, 1, 2, 4 * _D)


# speedup vs baseline: 1.5282x; 1.0502x over previous
import functools
import jax
import jax.numpy as jnp
from jax import lax
from jax.experimental import pallas as pl
from jax.experimental.pallas import tpu as pltpu
from jax.experimental.pallas import tpu_sc as plsc

_D = 256
_N_DRUG = 1373
_B = 16384
_NC = 2
_NS = 16
_NW = _NC * _NS
_CB = 32
_BPW = _B // _NW
_NCHUNK = _BPW // _CB
_L = 16


def _body(h1, h2, d1, d2, m1, m2, x1, x2, out,
          idx1_v, idx2_v, idx2h_v, big, sem):
    wid = lax.axis_index("s") * _NC + lax.axis_index("c")
    base_w = wid * _BPW

    pltpu.sync_copy(x1.at[pl.ds(base_w, _BPW)], idx1_v)
    pltpu.sync_copy(x2.at[pl.ds(base_w, _BPW)], idx2_v)
    for j in range(_BPW // _L):
        idx2h_v[pl.ds(j * _L, _L)] = idx2_v[pl.ds(j * _L, _L)] + _N_DRUG

    def chunk(c, _):
        off = c * _CB
        i1 = idx1_v.at[pl.ds(off, _CB)]
        i2 = idx2_v.at[pl.ds(off, _CB)]
        i2h = idx2h_v.at[pl.ds(off, _CB)]
        cps = [
            pltpu.async_copy(d1.at[i1], big.at[:, pl.ds(0 * _D, _D)], sem),
            pltpu.async_copy(d2.at[i1], big.at[:, pl.ds(1 * _D, _D)], sem),
            pltpu.async_copy(h1.at[i1], big.at[:, pl.ds(2 * _D, _D)], sem),
            pltpu.async_copy(h2.at[i1], big.at[:, pl.ds(3 * _D, _D)], sem),
            pltpu.async_copy(m1.at[i2], big.at[:, pl.ds(4 * _D, _D)], sem),
            pltpu.async_copy(m2.at[i2], big.at[:, pl.ds(5 * _D, _D)], sem),
            pltpu.async_copy(h1.at[i2h], big.at[:, pl.ds(6 * _D, _D)], sem),
            pltpu.async_copy(h2.at[i2h], big.at[:, pl.ds(7 * _D, _D)], sem),
        ]
        for cp in cps:
            cp.wait()
        pltpu.sync_copy(big, out.at[pl.ds(base_w + off, _CB), :])
        return ()

    lax.fori_loop(0, _NCHUNK, chunk, (), unroll=False)


@jax.jit
def _run(h1, h2, d1, d2, m1, m2, x1, x2):
    mesh = plsc.VectorSubcoreMesh(core_axis_name="c", subcore_axis_name="s")
    f = pl.kernel(
        _body,
        out_type=jax.ShapeDtypeStruct((_B, 8 * _D), jnp.float32),
        mesh=mesh,
        scratch_types=[
            pltpu.VMEM((_BPW,), jnp.int32),
            pltpu.VMEM((_BPW,), jnp.int32),
            pltpu.VMEM((_BPW,), jnp.int32),
            pltpu.VMEM((_CB, 8 * _D), jnp.float32),
            pltpu.SemaphoreType.DMA,
        ],
    )
    return f(h1, h2, d1, d2, m1, m2, x1, x2)


def kernel(hete_1hop, hete_2hop, drug_homo_1hop, drug_homo_2hop,
           mic_homo_1hop, mic_homo_2hop, x1, x2):
    out = _run(hete_1hop, hete_2hop, drug_homo_1hop, drug_homo_2hop,
               mic_homo_1hop, mic_homo_2hop,
               x1.astype(jnp.int32), x2.astype(jnp.int32))
    return out.reshape(_B, 1, 2, 4 * _D)


# speedup vs baseline: 1.5432x; 1.0098x over previous
import functools
import jax
import jax.numpy as jnp
from jax import lax
from jax.experimental import pallas as pl
from jax.experimental.pallas import tpu as pltpu
from jax.experimental.pallas import tpu_sc as plsc

_D = 256
_W = 8 * _D
_N_DRUG = 1373
_B = 16384
_NC = 2
_NS = 16
_NW = _NC * _NS
_CB = 16
_BPW = _B // _NW
_NCHUNK = _BPW // _CB
_L = 16


def _body(h1, h2, d1, d2, m1, m2, x1, x2, out,
          idx1_v, idx2_v, idx2h_v, bigA, bigB, gsA, gsB, wsA, wsB):
    wid = lax.axis_index("s") * _NC + lax.axis_index("c")
    base_w = wid * _BPW
    bufs = (bigA, bigB)
    gsems = (gsA, gsB)
    wsems = (wsA, wsB)

    pltpu.sync_copy(x1.at[pl.ds(base_w, _BPW)], idx1_v)
    pltpu.sync_copy(x2.at[pl.ds(base_w, _BPW)], idx2_v)
    for j in range(_BPW // _L):
        idx2h_v[pl.ds(j * _L, _L)] = idx2_v[pl.ds(j * _L, _L)] + _N_DRUG

    def fire_gathers(c, buf, sem):
        off = c * _CB
        i1 = idx1_v.at[pl.ds(off, _CB)]
        i2 = idx2_v.at[pl.ds(off, _CB)]
        i2h = idx2h_v.at[pl.ds(off, _CB)]
        for tab, idx, k in ((d1, i1, 0), (d2, i1, 1), (h1, i1, 2),
                            (h2, i1, 3), (m1, i2, 4), (m2, i2, 5),
                            (h1, i2h, 6), (h2, i2h, 7)):
            pltpu.async_copy(tab.at[idx], buf.at[:, pl.ds(k * _D, _D)], sem)

    def drain_gathers(buf, sem):
        pltpu.make_async_copy(out.at[pl.ds(0, _CB), :], buf, sem).wait()

    def fire_write(c, buf, sem):
        pltpu.async_copy(buf, out.at[pl.ds(base_w + c * _CB, _CB), :], sem)

    def drain_write(buf, sem):
        pltpu.make_async_copy(out.at[pl.ds(0, _CB), :], buf, sem).wait()

    fire_gathers(0, bufs[0], gsems[0])
    drain_gathers(bufs[0], gsems[0])
    fire_write(0, bufs[0], wsems[0])
    fire_gathers(1, bufs[1], gsems[1])

    def outer(o, _):
        for step in range(2):
            c = 2 * o + 1 + step
            x = (1 + step) % 2
            y = 1 - x
            drain_gathers(bufs[x], gsems[x])
            fire_write(c, bufs[x], wsems[x])
            drain_write(bufs[y], wsems[y])
            fire_gathers(c + 1, bufs[y], gsems[y])
        return ()

    lax.fori_loop(0, (_NCHUNK - 2) // 2, outer, (), unroll=False)

    cl = _NCHUNK - 1
    xl = cl % 2
    yl = 1 - xl
    drain_gathers(bufs[xl], gsems[xl])
    fire_write(cl, bufs[xl], wsems[xl])
    drain_write(bufs[yl], wsems[yl])
    drain_write(bufs[xl], wsems[xl])


@jax.jit
def _run(h1, h2, d1, d2, m1, m2, x1, x2):
    mesh = plsc.VectorSubcoreMesh(core_axis_name="c", subcore_axis_name="s")
    f = pl.kernel(
        _body,
        out_type=jax.ShapeDtypeStruct((_B, _W), jnp.float32),
        mesh=mesh,
        scratch_types=[
            pltpu.VMEM((_BPW,), jnp.int32),
            pltpu.VMEM((_BPW,), jnp.int32),
            pltpu.VMEM((_BPW,), jnp.int32),
            pltpu.VMEM((_CB, _W), jnp.float32),
            pltpu.VMEM((_CB, _W), jnp.float32),
            pltpu.SemaphoreType.DMA,
            pltpu.SemaphoreType.DMA,
            pltpu.SemaphoreType.DMA,
            pltpu.SemaphoreType.DMA,
        ],
    )
    return f(h1, h2, d1, d2, m1, m2, x1, x2)


def kernel(hete_1hop, hete_2hop, drug_homo_1hop, drug_homo_2hop,
           mic_homo_1hop, mic_homo_2hop, x1, x2):
    out = _run(hete_1hop, hete_2hop, drug_homo_1hop, drug_homo_2hop,
               mic_homo_1hop, mic_homo_2hop,
               x1.astype(jnp.int32), x2.astype(jnp.int32))
    return out.reshape(_B, 1, 2, 4 * _D)
